# baseline (speedup 1.0000x reference)
import functools

import jax
import jax.numpy as jnp
from jax import lax
from jax.experimental import pallas as pl
from jax.experimental.pallas import tpu as pltpu
from jax.experimental.pallas import tpu_sc as plsc

T = 2048
D_MODEL = 1024
D_HIDDEN = 2048
E = 16
K = 3
BLK = 256
NBUF = 10240
NBLOCKS = NBUF // BLK
CHK = 128
P = K * T
TB = 256

NW = 32
ROWS_PER_W = T // NW


def _route_body(u_ref, gw_ref, dest_ref, scores_ref, map_ref):
    x = u_ref[...]
    logits = lax.dot_general(x, gw_ref[...], (((1,), (0,)), ((), ())),
                             preferred_element_type=jnp.float32)
    eiota = lax.broadcasted_iota(jnp.int32, (T, E), 1)
    neg = jnp.float32(-1e30)

    cur = logits
    ids = []
    vals = []
    for _ in range(K):
        m = jnp.max(cur, axis=1, keepdims=True)
        amax = jnp.min(jnp.where(cur == m, eiota, E), axis=1,
                       keepdims=True)
        ids.append(amax)
        vals.append(m)
        cur = jnp.where(eiota == amax, neg, cur)

    vmax = vals[0]
    exps = [jnp.exp(v - vmax) for v in vals]
    denom = exps[0]
    for k in range(1, K):
        denom = denom + exps[k]

    onehots = [(eiota == i).astype(jnp.float32) for i in ids]

    li = lax.broadcasted_iota(jnp.int32, (CHK, CHK), 0)
    lj = lax.broadcasted_iota(jnp.int32, (CHK, CHK), 1)
    lstrict = (lj < li).astype(jnp.float32)

    running = jnp.zeros((1, E), jnp.float32)
    rank_cols = []
    for k in range(K):
        oh = onehots[k]
        chunks = []
        for c in range(T // CHK):
            blk = oh[c * CHK:(c + 1) * CHK]
            intra = lax.dot_general(lstrict, blk, (((1,), (0,)), ((), ())),
                                    preferred_element_type=jnp.float32)
            chunks.append(intra + running)
            running = running + jnp.sum(blk, axis=0, keepdims=True)
        rk = jnp.concatenate(chunks, axis=0)
        rank_cols.append(jnp.sum(rk * oh, axis=1, keepdims=True))

    counts = running
    padded = jnp.ceil(counts / BLK) * BLK
    ei = lax.broadcasted_iota(jnp.int32, (E, E), 0)
    ej = lax.broadcasted_iota(jnp.int32, (E, E), 1)
    ustrict = (ei < ej).astype(jnp.float32)
    offsets = lax.dot_general(padded, ustrict, (((1,), (0,)), ((), ())),
                              preferred_element_type=jnp.float32)

    for k in range(K):
        off_k = jnp.sum(onehots[k] * offsets, axis=1, keepdims=True)
        dest_k = off_k + rank_cols[k]
        dest_ref[k, :] = dest_k[:, 0].astype(jnp.int32)
        scores_ref[k, :] = (exps[k] / denom)[:, 0]

    b0 = lax.broadcasted_iota(jnp.int32, (NBLOCKS, 1), 0).astype(jnp.float32)
    b0 = b0 * BLK
    inb = ((b0 >= offsets) & (b0 < offsets + padded)).astype(jnp.float32)
    eidx = lax.broadcasted_iota(jnp.int32, (NBLOCKS, E), 1).astype(jnp.float32)
    map_ref[0, :] = jnp.sum(inb * eidx, axis=1).astype(jnp.int32)


def _route(u, gate_w):
    return pl.pallas_call(
        _route_body,
        out_shape=[
            jax.ShapeDtypeStruct((K, T), jnp.int32),
            jax.ShapeDtypeStruct((K, T), jnp.float32),
            jax.ShapeDtypeStruct((1, NBLOCKS), jnp.int32),
        ],
    )(u, gate_w)


def _shared_body(u_ref, w1_ref, b1_ref, w2_ref, b2_ref, o_ref):
    x = u_ref[...]
    xb = x.astype(jnp.bfloat16)
    h = lax.dot_general(xb, w1_ref[0], (((1,), (1,)), ((), ())),
                        preferred_element_type=jnp.float32) + b1_ref[...]
    h = jnp.maximum(h, 0.0).astype(jnp.bfloat16)
    y = lax.dot_general(h, w2_ref[0], (((1,), (1,)), ((), ())),
                        preferred_element_type=jnp.float32) + b2_ref[...]
    o_ref[...] = x + y


def _shared(u, w1, b1, w2, b2):
    return pl.pallas_call(
        _shared_body,
        grid=(T // TB,),
        in_specs=[
            pl.BlockSpec((TB, D_MODEL), lambda i: (i, 0)),
            pl.BlockSpec((1, D_HIDDEN, D_MODEL), lambda i: (0, 0, 0)),
            pl.BlockSpec((1, D_HIDDEN), lambda i: (0, 0)),
            pl.BlockSpec((1, D_MODEL, D_HIDDEN), lambda i: (0, 0, 0)),
            pl.BlockSpec((1, D_MODEL), lambda i: (0, 0)),
        ],
        out_specs=pl.BlockSpec((TB, D_MODEL), lambda i: (i, 0)),
        out_shape=jax.ShapeDtypeStruct((T, D_MODEL), jnp.float32),
    )(u, w1, b1, w2, b2)


def _sc_scatter(u, dest):
    mesh = plsc.VectorSubcoreMesh(core_axis_name="c", subcore_axis_name="s")

    @functools.partial(
        pl.kernel, mesh=mesh,
        out_type=jax.ShapeDtypeStruct((NBUF, D_MODEL), jnp.float32),
        scratch_types=[
            pltpu.VMEM((ROWS_PER_W,), jnp.int32),
            pltpu.VMEM((ROWS_PER_W, D_MODEL), jnp.float32),
            pltpu.SemaphoreType.DMA,
        ],
    )
    def body(u_hbm, dest_hbm, xs_hbm, idx_v, rows_v, sem):
        wid = lax.axis_index("s") * 2 + lax.axis_index("c")
        base_t = wid * ROWS_PER_W
        pltpu.sync_copy(u_hbm.at[pl.ds(base_t, ROWS_PER_W)], rows_v)
        for k in range(K):
            pltpu.sync_copy(dest_hbm.at[pl.ds(k * T + base_t, ROWS_PER_W)],
                            idx_v)
            pltpu.async_copy(rows_v, xs_hbm.at[idx_v], sem).wait()

    return body(u, dest)


def _sc_gather(y_sorted, dest):
    mesh = plsc.VectorSubcoreMesh(core_axis_name="c", subcore_axis_name="s")

    @functools.partial(
        pl.kernel, mesh=mesh,
        out_type=jax.ShapeDtypeStruct((P, D_MODEL), jnp.float32),
        scratch_types=[
            pltpu.VMEM((ROWS_PER_W,), jnp.int32),
            pltpu.VMEM((ROWS_PER_W, D_MODEL), jnp.float32),
            pltpu.SemaphoreType.DMA,
        ],
    )
    def body(ys_hbm, dest_hbm, out_hbm, idx_v, rows_v, sem):
        wid = lax.axis_index("s") * 2 + lax.axis_index("c")
        base_t = wid * ROWS_PER_W
        for k in range(K):
            p0 = k * T + base_t
            pltpu.sync_copy(dest_hbm.at[pl.ds(p0, ROWS_PER_W)], idx_v)
            pltpu.async_copy(ys_hbm.at[idx_v], rows_v, sem).wait()
            pltpu.sync_copy(rows_v, out_hbm.at[pl.ds(p0, ROWS_PER_W)])

    return body(y_sorted, dest)


def _group_body(map_ref, x_ref, w1_ref, b1_ref, w2_ref, b2_ref, y_ref):
    x = x_ref[...].astype(jnp.bfloat16)
    h = lax.dot_general(x, w1_ref[0], (((1,), (1,)), ((), ())),
                        preferred_element_type=jnp.float32) + b1_ref[0]
    h = jnp.maximum(h, 0.0).astype(jnp.bfloat16)
    y_ref[...] = lax.dot_general(h, w2_ref[0], (((1,), (1,)), ((), ())),
                                 preferred_element_type=jnp.float32) + b2_ref[0]


def _grouped_ffn(blkmap, x_sorted, w1, b1, w2, b2):
    grid_spec = pltpu.PrefetchScalarGridSpec(
        num_scalar_prefetch=1,
        grid=(NBLOCKS,),
        in_specs=[
            pl.BlockSpec((BLK, D_MODEL), lambda b, m: (b, 0)),
            pl.BlockSpec((1, D_HIDDEN, D_MODEL), lambda b, m: (m[b], 0, 0)),
            pl.BlockSpec((1, 1, D_HIDDEN), lambda b, m: (m[b], 0, 0)),
            pl.BlockSpec((1, D_MODEL, D_HIDDEN), lambda b, m: (m[b], 0, 0)),
            pl.BlockSpec((1, 1, D_MODEL), lambda b, m: (m[b], 0, 0)),
        ],
        out_specs=pl.BlockSpec((BLK, D_MODEL), lambda b, m: (b, 0)),
    )
    return pl.pallas_call(
        _group_body,
        grid_spec=grid_spec,
        out_shape=jax.ShapeDtypeStruct((NBUF, D_MODEL), jnp.float32),
    )(blkmap, x_sorted, w1, b1.reshape(E, 1, D_HIDDEN), w2,
      b2.reshape(E, 1, D_MODEL))


def _combine_body(base_ref, yg_ref, sc_ref, o_ref):
    acc = base_ref[...]
    for k in range(K):
        acc = acc + sc_ref[k, :][:, None] * yg_ref[k]
    o_ref[...] = acc


def _combine(out_base, y_gath, scores):
    return pl.pallas_call(
        _combine_body,
        grid=(T // TB,),
        in_specs=[
            pl.BlockSpec((TB, D_MODEL), lambda i: (i, 0)),
            pl.BlockSpec((K, TB, D_MODEL), lambda i: (0, i, 0)),
            pl.BlockSpec((K, TB), lambda i: (0, i)),
        ],
        out_specs=pl.BlockSpec((TB, D_MODEL), lambda i: (i, 0)),
        out_shape=jax.ShapeDtypeStruct((T, D_MODEL), jnp.float32),
    )(out_base, y_gath, scores)


def kernel(u, gate_w, s_fc1_w, s_fc1_b, s_fc2_w, s_fc2_b,
           r_fc1_w, r_fc1_b, r_fc2_w, r_fc2_b):
    dest_kt, scores_kt, blkmap = _route(u, gate_w)
    dest = dest_kt.reshape(P)
    blkmap = blkmap.reshape(NBLOCKS)

    out_base = _shared(u, s_fc1_w.astype(jnp.bfloat16), s_fc1_b,
                       s_fc2_w.astype(jnp.bfloat16), s_fc2_b)
    x_sorted = _sc_scatter(u, dest)
    y_sorted = _grouped_ffn(blkmap, x_sorted,
                            r_fc1_w.astype(jnp.bfloat16), r_fc1_b,
                            r_fc2_w.astype(jnp.bfloat16), r_fc2_b)
    y_gath = _sc_gather(y_sorted, dest).reshape(K, T, D_MODEL)
    return _combine(out_base, y_gath, scores_kt)


# speedup vs baseline: 1.4334x; 1.4334x over previous
import functools

import jax
import jax.numpy as jnp
from jax import lax
from jax.experimental import pallas as pl
from jax.experimental.pallas import tpu as pltpu
from jax.experimental.pallas import tpu_sc as plsc

T = 2048
D_MODEL = 1024
D_HIDDEN = 2048
E = 16
K = 3
BLK = 256
NBUF = 10240
NBLOCKS = NBUF // BLK
CHK = 128
P = K * T
TB = 256

NW = 32
ROWS_PER_W = T // NW


def _route_body(u_ref, gw_ref, dest_ref, scores_ref, map_ref, nused_ref):
    x = u_ref[...]
    logits = lax.dot_general(x, gw_ref[...], (((1,), (0,)), ((), ())),
                             preferred_element_type=jnp.float32)
    eiota = lax.broadcasted_iota(jnp.int32, (T, E), 1)
    neg = jnp.float32(-1e30)

    cur = logits
    ids = []
    vals = []
    for _ in range(K):
        m = jnp.max(cur, axis=1, keepdims=True)
        amax = jnp.min(jnp.where(cur == m, eiota, E), axis=1,
                       keepdims=True)
        ids.append(amax)
        vals.append(m)
        cur = jnp.where(eiota == amax, neg, cur)

    vmax = vals[0]
    exps = [jnp.exp(v - vmax) for v in vals]
    denom = exps[0]
    for k in range(1, K):
        denom = denom + exps[k]

    onehots = [(eiota == i).astype(jnp.float32) for i in ids]

    li = lax.broadcasted_iota(jnp.int32, (CHK, CHK), 0)
    lj = lax.broadcasted_iota(jnp.int32, (CHK, CHK), 1)
    lstrict = (lj < li).astype(jnp.float32)

    running = jnp.zeros((1, E), jnp.float32)
    rank_cols = []
    for k in range(K):
        oh = onehots[k]
        chunks = []
        for c in range(T // CHK):
            blk = oh[c * CHK:(c + 1) * CHK]
            intra = lax.dot_general(lstrict, blk, (((1,), (0,)), ((), ())),
                                    preferred_element_type=jnp.float32)
            chunks.append(intra + running)
            running = running + jnp.sum(blk, axis=0, keepdims=True)
        rk = jnp.concatenate(chunks, axis=0)
        rank_cols.append(jnp.sum(rk * oh, axis=1, keepdims=True))

    counts = running
    padded = jnp.ceil(counts / BLK) * BLK
    ei = lax.broadcasted_iota(jnp.int32, (E, E), 0)
    ej = lax.broadcasted_iota(jnp.int32, (E, E), 1)
    ustrict = (ei < ej).astype(jnp.float32)
    offsets = lax.dot_general(padded, ustrict, (((1,), (0,)), ((), ())),
                              preferred_element_type=jnp.float32)

    for k in range(K):
        off_k = jnp.sum(onehots[k] * offsets, axis=1, keepdims=True)
        dest_k = off_k + rank_cols[k]
        dest_ref[k, :] = dest_k[:, 0].astype(jnp.int32)
        scores_ref[k, :] = (exps[k] / denom)[:, 0]

    b0 = lax.broadcasted_iota(jnp.int32, (NBLOCKS, 1), 0).astype(jnp.float32)
    b0 = b0 * BLK
    inb = (b0 >= offsets).astype(jnp.float32)
    map_ref[0, :] = (jnp.sum(inb, axis=1) - 1.0).astype(jnp.int32)
    nused_ref[...] = (jnp.sum(padded, axis=1, keepdims=True) / BLK).astype(jnp.int32)


def _route(u, gate_w):
    return pl.pallas_call(
        _route_body,
        out_shape=[
            jax.ShapeDtypeStruct((K, T), jnp.int32),
            jax.ShapeDtypeStruct((K, T), jnp.float32),
            jax.ShapeDtypeStruct((1, NBLOCKS), jnp.int32),
            jax.ShapeDtypeStruct((1, 1), jnp.int32),
        ],
    )(u, gate_w)


def _shared_body(u_ref, w1_ref, b1_ref, w2_ref, b2_ref, o_ref):
    x = u_ref[...]
    h = lax.dot_general(x, w1_ref[0], (((1,), (1,)), ((), ())),
                        preferred_element_type=jnp.float32) + b1_ref[...]
    h = jnp.maximum(h, 0.0)
    y = lax.dot_general(h, w2_ref[0], (((1,), (1,)), ((), ())),
                        preferred_element_type=jnp.float32) + b2_ref[...]
    o_ref[...] = x + y


def _shared(u, w1, b1, w2, b2):
    return pl.pallas_call(
        _shared_body,
        grid=(T // TB,),
        in_specs=[
            pl.BlockSpec((TB, D_MODEL), lambda i: (i, 0)),
            pl.BlockSpec((1, D_HIDDEN, D_MODEL), lambda i: (0, 0, 0)),
            pl.BlockSpec((1, D_HIDDEN), lambda i: (0, 0)),
            pl.BlockSpec((1, D_MODEL, D_HIDDEN), lambda i: (0, 0, 0)),
            pl.BlockSpec((1, D_MODEL), lambda i: (0, 0)),
        ],
        out_specs=pl.BlockSpec((TB, D_MODEL), lambda i: (i, 0)),
        out_shape=jax.ShapeDtypeStruct((T, D_MODEL), jnp.float32),
    )(u, w1, b1, w2, b2)


def _sc_scatter(u, dest):
    mesh = plsc.VectorSubcoreMesh(core_axis_name="c", subcore_axis_name="s")

    @functools.partial(
        pl.kernel, mesh=mesh,
        out_type=jax.ShapeDtypeStruct((NBUF, D_MODEL), jnp.float32),
        scratch_types=[
            pltpu.VMEM((ROWS_PER_W,), jnp.int32),
            pltpu.VMEM((ROWS_PER_W, D_MODEL), jnp.float32),
            pltpu.SemaphoreType.DMA,
        ],
    )
    def body(u_hbm, dest_hbm, xs_hbm, idx_v, rows_v, sem):
        wid = lax.axis_index("s") * 2 + lax.axis_index("c")
        base_t = wid * ROWS_PER_W
        pltpu.sync_copy(u_hbm.at[pl.ds(base_t, ROWS_PER_W)], rows_v)
        for k in range(K):
            pltpu.sync_copy(dest_hbm.at[pl.ds(k * T + base_t, ROWS_PER_W)],
                            idx_v)
            pltpu.async_copy(rows_v, xs_hbm.at[idx_v], sem).wait()

    return body(u, dest)


def _sc_gather(y_sorted, dest):
    mesh = plsc.VectorSubcoreMesh(core_axis_name="c", subcore_axis_name="s")

    @functools.partial(
        pl.kernel, mesh=mesh,
        out_type=jax.ShapeDtypeStruct((P, D_MODEL), jnp.float32),
        scratch_types=[
            pltpu.VMEM((ROWS_PER_W,), jnp.int32),
            pltpu.VMEM((ROWS_PER_W, D_MODEL), jnp.float32),
            pltpu.SemaphoreType.DMA,
        ],
    )
    def body(ys_hbm, dest_hbm, out_hbm, idx_v, rows_v, sem):
        wid = lax.axis_index("s") * 2 + lax.axis_index("c")
        base_t = wid * ROWS_PER_W
        for k in range(K):
            p0 = k * T + base_t
            pltpu.sync_copy(dest_hbm.at[pl.ds(p0, ROWS_PER_W)], idx_v)
            pltpu.async_copy(ys_hbm.at[idx_v], rows_v, sem).wait()
            pltpu.sync_copy(rows_v, out_hbm.at[pl.ds(p0, ROWS_PER_W)])

    return body(y_sorted, dest)


def _group_body(map_ref, x_ref, w1_ref, b1_ref, w2_ref, b2_ref, y_ref):
    @pl.when(pl.program_id(0) < map_ref[NBLOCKS])
    def _():
        x = x_ref[...]
        h = lax.dot_general(x, w1_ref[0], (((1,), (1,)), ((), ())),
                            preferred_element_type=jnp.float32) + b1_ref[0]
        h = jnp.maximum(h, 0.0)
        y_ref[...] = lax.dot_general(h, w2_ref[0], (((1,), (1,)), ((), ())),
                                     preferred_element_type=jnp.float32) + b2_ref[0]


def _grouped_ffn(blkmap, x_sorted, w1, b1, w2, b2):
    grid_spec = pltpu.PrefetchScalarGridSpec(
        num_scalar_prefetch=1,
        grid=(NBLOCKS,),
        in_specs=[
            pl.BlockSpec((BLK, D_MODEL), lambda b, m: (b, 0)),
            pl.BlockSpec((1, D_HIDDEN, D_MODEL), lambda b, m: (m[b], 0, 0)),
            pl.BlockSpec((1, 1, D_HIDDEN), lambda b, m: (m[b], 0, 0)),
            pl.BlockSpec((1, D_MODEL, D_HIDDEN), lambda b, m: (m[b], 0, 0)),
            pl.BlockSpec((1, 1, D_MODEL), lambda b, m: (m[b], 0, 0)),
        ],
        out_specs=pl.BlockSpec((BLK, D_MODEL), lambda b, m: (b, 0)),
    )
    return pl.pallas_call(
        _group_body,
        grid_spec=grid_spec,
        out_shape=jax.ShapeDtypeStruct((NBUF, D_MODEL), jnp.float32),
    )(blkmap, x_sorted, w1, b1.reshape(E, 1, D_HIDDEN), w2,
      b2.reshape(E, 1, D_MODEL))


def _combine_body(base_ref, yg_ref, sc_ref, o_ref):
    acc = base_ref[...]
    for k in range(K):
        acc = acc + sc_ref[k, :][:, None] * yg_ref[k]
    o_ref[...] = acc


def _combine(out_base, y_gath, scores):
    return pl.pallas_call(
        _combine_body,
        grid=(T // TB,),
        in_specs=[
            pl.BlockSpec((TB, D_MODEL), lambda i: (i, 0)),
            pl.BlockSpec((K, TB, D_MODEL), lambda i: (0, i, 0)),
            pl.BlockSpec((K, TB), lambda i: (0, i)),
        ],
        out_specs=pl.BlockSpec((TB, D_MODEL), lambda i: (i, 0)),
        out_shape=jax.ShapeDtypeStruct((T, D_MODEL), jnp.float32),
    )(out_base, y_gath, scores)


def kernel(u, gate_w, s_fc1_w, s_fc1_b, s_fc2_w, s_fc2_b,
           r_fc1_w, r_fc1_b, r_fc2_w, r_fc2_b):
    dest_kt, scores_kt, blkmap, nused = _route(u, gate_w)
    dest = dest_kt.reshape(P)
    blkmap = jnp.concatenate([blkmap.reshape(NBLOCKS), nused.reshape(1)])

    out_base = _shared(u, s_fc1_w, s_fc1_b, s_fc2_w, s_fc2_b)
    x_sorted = _sc_scatter(u, dest)
    y_sorted = _grouped_ffn(blkmap, x_sorted, r_fc1_w, r_fc1_b, r_fc2_w, r_fc2_b)
    y_gath = _sc_gather(y_sorted, dest).reshape(K, T, D_MODEL)
    return _combine(out_base, y_gath, scores_kt)


# speedup vs baseline: 1.6978x; 1.1845x over previous
import functools

import jax
import jax.numpy as jnp
from jax import lax
from jax.experimental import pallas as pl
from jax.experimental.pallas import tpu as pltpu
from jax.experimental.pallas import tpu_sc as plsc

T = 2048
D_MODEL = 1024
D_HIDDEN = 2048
E = 16
K = 3
BLK = 512
NBUF = 14336
NBLOCKS = NBUF // BLK
CHK = 128
P = K * T
TB = 256

NW = 32
ROWS_PER_W = T // NW


def _route_body(u_ref, gw_ref, dest_ref, scores_ref, map_ref, nused_ref):
    x = u_ref[...]
    logits = lax.dot_general(x, gw_ref[...], (((1,), (0,)), ((), ())),
                             preferred_element_type=jnp.float32)
    eiota = lax.broadcasted_iota(jnp.int32, (T, E), 1)
    neg = jnp.float32(-1e30)

    cur = logits
    ids = []
    vals = []
    for _ in range(K):
        m = jnp.max(cur, axis=1, keepdims=True)
        amax = jnp.min(jnp.where(cur == m, eiota, E), axis=1,
                       keepdims=True)
        ids.append(amax)
        vals.append(m)
        cur = jnp.where(eiota == amax, neg, cur)

    vmax = vals[0]
    exps = [jnp.exp(v - vmax) for v in vals]
    denom = exps[0]
    for k in range(1, K):
        denom = denom + exps[k]

    onehots = [(eiota == i).astype(jnp.float32) for i in ids]

    li = lax.broadcasted_iota(jnp.int32, (CHK, CHK), 0)
    lj = lax.broadcasted_iota(jnp.int32, (CHK, CHK), 1)
    lstrict = (lj < li).astype(jnp.float32)

    running = jnp.zeros((1, E), jnp.float32)
    rank_cols = []
    for k in range(K):
        oh = onehots[k]
        chunks = []
        for c in range(T // CHK):
            blk = oh[c * CHK:(c + 1) * CHK]
            intra = lax.dot_general(lstrict, blk, (((1,), (0,)), ((), ())),
                                    preferred_element_type=jnp.float32)
            chunks.append(intra + running)
            running = running + jnp.sum(blk, axis=0, keepdims=True)
        rk = jnp.concatenate(chunks, axis=0)
        rank_cols.append(jnp.sum(rk * oh, axis=1, keepdims=True))

    counts = running
    padded = jnp.ceil(counts / BLK) * BLK
    ei = lax.broadcasted_iota(jnp.int32, (E, E), 0)
    ej = lax.broadcasted_iota(jnp.int32, (E, E), 1)
    ustrict = (ei < ej).astype(jnp.float32)
    offsets = lax.dot_general(padded, ustrict, (((1,), (0,)), ((), ())),
                              preferred_element_type=jnp.float32)

    for k in range(K):
        off_k = jnp.sum(onehots[k] * offsets, axis=1, keepdims=True)
        dest_k = off_k + rank_cols[k]
        dest_ref[k, :] = dest_k[:, 0].astype(jnp.int32)
        scores_ref[k, :] = (exps[k] / denom)[:, 0]

    b0 = lax.broadcasted_iota(jnp.int32, (NBLOCKS, 1), 0).astype(jnp.float32)
    b0 = b0 * BLK
    inb = (b0 >= offsets).astype(jnp.float32)
    map_ref[0, :] = (jnp.sum(inb, axis=1) - 1.0).astype(jnp.int32)
    nused_ref[...] = (jnp.sum(padded, axis=1, keepdims=True) / BLK).astype(jnp.int32)


def _route(u, gate_w):
    return pl.pallas_call(
        _route_body,
        out_shape=[
            jax.ShapeDtypeStruct((K, T), jnp.int32),
            jax.ShapeDtypeStruct((K, T), jnp.float32),
            jax.ShapeDtypeStruct((1, NBLOCKS), jnp.int32),
            jax.ShapeDtypeStruct((1, 1), jnp.int32),
        ],
    )(u, gate_w)


def _shared_body(u_ref, w1_ref, b1_ref, w2_ref, b2_ref, o_ref):
    x = u_ref[...]
    h = lax.dot_general(x, w1_ref[0], (((1,), (1,)), ((), ())),
                        preferred_element_type=jnp.float32) + b1_ref[...]
    h = jnp.maximum(h, 0.0)
    y = lax.dot_general(h, w2_ref[0], (((1,), (1,)), ((), ())),
                        preferred_element_type=jnp.float32) + b2_ref[...]
    o_ref[...] = x + y


def _shared(u, w1, b1, w2, b2):
    return pl.pallas_call(
        _shared_body,
        grid=(T // TB,),
        in_specs=[
            pl.BlockSpec((TB, D_MODEL), lambda i: (i, 0)),
            pl.BlockSpec((1, D_HIDDEN, D_MODEL), lambda i: (0, 0, 0)),
            pl.BlockSpec((1, D_HIDDEN), lambda i: (0, 0)),
            pl.BlockSpec((1, D_MODEL, D_HIDDEN), lambda i: (0, 0, 0)),
            pl.BlockSpec((1, D_MODEL), lambda i: (0, 0)),
        ],
        out_specs=pl.BlockSpec((TB, D_MODEL), lambda i: (i, 0)),
        out_shape=jax.ShapeDtypeStruct((T, D_MODEL), jnp.float32),
    )(u, w1, b1, w2, b2)


def _sc_scatter(u, dest):
    mesh = plsc.VectorSubcoreMesh(core_axis_name="c", subcore_axis_name="s")

    @functools.partial(
        pl.kernel, mesh=mesh,
        out_type=jax.ShapeDtypeStruct((NBUF, D_MODEL), jnp.float32),
        scratch_types=[
            pltpu.VMEM((ROWS_PER_W,), jnp.int32),
            pltpu.VMEM((ROWS_PER_W, D_MODEL), jnp.float32),
            pltpu.SemaphoreType.DMA,
        ],
    )
    def body(u_hbm, dest_hbm, xs_hbm, idx_v, rows_v, sem):
        wid = lax.axis_index("s") * 2 + lax.axis_index("c")
        base_t = wid * ROWS_PER_W
        pltpu.sync_copy(u_hbm.at[pl.ds(base_t, ROWS_PER_W)], rows_v)
        for k in range(K):
            pltpu.sync_copy(dest_hbm.at[pl.ds(k * T + base_t, ROWS_PER_W)],
                            idx_v)
            pltpu.async_copy(rows_v, xs_hbm.at[idx_v], sem).wait()

    return body(u, dest)


def _sc_gather(y_sorted, dest):
    mesh = plsc.VectorSubcoreMesh(core_axis_name="c", subcore_axis_name="s")

    @functools.partial(
        pl.kernel, mesh=mesh,
        out_type=jax.ShapeDtypeStruct((P, D_MODEL), jnp.float32),
        scratch_types=[
            pltpu.VMEM((ROWS_PER_W,), jnp.int32),
            pltpu.VMEM((ROWS_PER_W, D_MODEL), jnp.float32),
            pltpu.SemaphoreType.DMA,
        ],
    )
    def body(ys_hbm, dest_hbm, out_hbm, idx_v, rows_v, sem):
        wid = lax.axis_index("s") * 2 + lax.axis_index("c")
        base_t = wid * ROWS_PER_W
        for k in range(K):
            p0 = k * T + base_t
            pltpu.sync_copy(dest_hbm.at[pl.ds(p0, ROWS_PER_W)], idx_v)
            pltpu.async_copy(ys_hbm.at[idx_v], rows_v, sem).wait()
            pltpu.sync_copy(rows_v, out_hbm.at[pl.ds(p0, ROWS_PER_W)])

    return body(y_sorted, dest)


def _group_body(map_ref, x_ref, w1_ref, b1_ref, w2_ref, b2_ref, y_ref):
    @pl.when(pl.program_id(0) < map_ref[NBLOCKS])
    def _():
        x = x_ref[...]
        h = lax.dot_general(x, w1_ref[0], (((1,), (1,)), ((), ())),
                            preferred_element_type=jnp.float32) + b1_ref[0]
        h = jnp.maximum(h, 0.0)
        y_ref[...] = lax.dot_general(h, w2_ref[0], (((1,), (1,)), ((), ())),
                                     preferred_element_type=jnp.float32) + b2_ref[0]


def _grouped_ffn(blkmap, x_sorted, w1, b1, w2, b2):
    grid_spec = pltpu.PrefetchScalarGridSpec(
        num_scalar_prefetch=1,
        grid=(NBLOCKS,),
        in_specs=[
            pl.BlockSpec((BLK, D_MODEL), lambda b, m: (b, 0)),
            pl.BlockSpec((1, D_HIDDEN, D_MODEL), lambda b, m: (m[b], 0, 0)),
            pl.BlockSpec((1, 1, D_HIDDEN), lambda b, m: (m[b], 0, 0)),
            pl.BlockSpec((1, D_MODEL, D_HIDDEN), lambda b, m: (m[b], 0, 0)),
            pl.BlockSpec((1, 1, D_MODEL), lambda b, m: (m[b], 0, 0)),
        ],
        out_specs=pl.BlockSpec((BLK, D_MODEL), lambda b, m: (b, 0)),
    )
    return pl.pallas_call(
        _group_body,
        grid_spec=grid_spec,
        out_shape=jax.ShapeDtypeStruct((NBUF, D_MODEL), jnp.float32),
    )(blkmap, x_sorted, w1, b1.reshape(E, 1, D_HIDDEN), w2,
      b2.reshape(E, 1, D_MODEL))


def _combine_body(base_ref, yg_ref, sc_ref, o_ref):
    acc = base_ref[...]
    for k in range(K):
        acc = acc + sc_ref[k, :][:, None] * yg_ref[k]
    o_ref[...] = acc


def _combine(out_base, y_gath, scores):
    return pl.pallas_call(
        _combine_body,
        grid=(T // TB,),
        in_specs=[
            pl.BlockSpec((TB, D_MODEL), lambda i: (i, 0)),
            pl.BlockSpec((K, TB, D_MODEL), lambda i: (0, i, 0)),
            pl.BlockSpec((K, TB), lambda i: (0, i)),
        ],
        out_specs=pl.BlockSpec((TB, D_MODEL), lambda i: (i, 0)),
        out_shape=jax.ShapeDtypeStruct((T, D_MODEL), jnp.float32),
    )(out_base, y_gath, scores)


def kernel(u, gate_w, s_fc1_w, s_fc1_b, s_fc2_w, s_fc2_b,
           r_fc1_w, r_fc1_b, r_fc2_w, r_fc2_b):
    dest_kt, scores_kt, blkmap, nused = _route(u, gate_w)
    dest = dest_kt.reshape(P)
    blkmap = jnp.concatenate([blkmap.reshape(NBLOCKS), nused.reshape(1)])

    out_base = _shared(u, s_fc1_w, s_fc1_b, s_fc2_w, s_fc2_b)
    x_sorted = _sc_scatter(u, dest)
    y_sorted = _grouped_ffn(blkmap, x_sorted, r_fc1_w, r_fc1_b, r_fc2_w, r_fc2_b)
    y_gath = _sc_gather(y_sorted, dest).reshape(K, T, D_MODEL)
    return _combine(out_base, y_gath, scores_kt)
